# baseline (speedup 1.0000x reference)
import functools

import jax
import jax.numpy as jnp
from jax import lax
from jax.experimental import pallas as pl
from jax.experimental.pallas import tpu as pltpu, tpu_sc as plsc

N = 10000
E = 320000
G = 64
N_PAD = 10240
NC = 2
NS = 16
CH = 80
EPT = E // NS
CHP = 80
NCHP = 250
NCHD = E // (NC * NS) // CH
NBUF = 5
ROWS_PER_TILE = N_PAD // NS
BR = 256
F32 = jnp.float32


def _mesh():
    return plsc.VectorSubcoreMesh(core_axis_name="c", subcore_axis_name="s")


def _zero_fill(ref, rows, d):
    per_row = d // 16

    def body(k, _):
        i = k // per_row
        j = k % per_row
        ref[i, pl.ds(j * 16, 16)] = jnp.zeros((16,), F32)
        return 0

    lax.fori_loop(0, rows * per_row, body, 0)


def _make_seg_sum(d):
    hd = d // 2
    zrows = 64
    nz = ROWS_PER_TILE // zrows
    nbuf = {64: 5, 32: 25, 16: 25}[hd]

    @functools.partial(
        pl.kernel,
        mesh=_mesh(),
        compiler_params=pltpu.CompilerParams(use_tc_tiling_on_sc=False),
        out_type=jax.ShapeDtypeStruct((NC, N_PAD, hd), F32),
        scratch_types=[
            pltpu.VMEM((NCHP, CHP), jnp.int32),
            pltpu.VMEM((NCHP, CHP), jnp.int32),
            pltpu.VMEM((nbuf * CHP, hd), F32),
            pltpu.VMEM((zrows, hd), F32),
            pltpu.VMEM_SHARED((N_PAD, hd), F32),
            pltpu.SemaphoreType.DMA,
            pltpu.SemaphoreType.DMA,
        ],
    )
    def seg(zs2_hbm, edges_hbm, out_hbm, src_v, dst_v, rows_v, zbuf, acc,
            gsem, ssem):
        cid = lax.axis_index("c")
        sid = lax.axis_index("s")
        row0 = sid * ROWS_PER_TILE

        _zero_fill(zbuf, zrows, d // 2)
        for i in range(nz):
            pltpu.sync_copy(zbuf, acc.at[pl.ds(row0 + i * zrows, zrows)])
        pltpu.sync_copy(edges_hbm.at[0, sid], src_v)
        pltpu.sync_copy(edges_hbm.at[1, sid], dst_v)

        plsc.subcore_barrier()

        zsrc = zs2_hbm.at[cid]

        def group(g, _):
            j0 = g * nbuf
            for b in range(nbuf):
                pltpu.async_copy(zsrc.at[src_v.at[j0 + b]],
                                 rows_v.at[pl.ds(b * CHP, CHP)], gsem)
            pltpu.make_async_copy(zsrc.at[pl.ds(0, nbuf * CHP)], rows_v,
                                  gsem).wait()
            for b in range(nbuf):
                pltpu.async_copy(rows_v.at[pl.ds(b * CHP, CHP)],
                                 acc.at[dst_v.at[j0 + b]], ssem, add=True)
            pltpu.make_async_copy(rows_v, acc.at[pl.ds(0, nbuf * CHP)],
                                  ssem).wait()
            return 0

        lax.fori_loop(0, NCHP // nbuf, group, 0)

        plsc.subcore_barrier()
        pltpu.sync_copy(acc.at[pl.ds(row0, ROWS_PER_TILE)],
                        out_hbm.at[cid, pl.ds(row0, ROWS_PER_TILE)])

    return seg


def _make_deg():
    @functools.partial(
        pl.kernel,
        mesh=_mesh(),
        compiler_params=pltpu.CompilerParams(use_tc_tiling_on_sc=False),
        out_type=jax.ShapeDtypeStruct((NC, N_PAD), F32),
        scratch_types=[
            pltpu.VMEM((NCHD, CH), jnp.int32),
            pltpu.VMEM((CH,), F32),
            pltpu.VMEM((ROWS_PER_TILE,), F32),
            pltpu.VMEM_SHARED((N_PAD,), F32),
            pltpu.SemaphoreType.DMA((NBUF,)),
        ],
    )
    def deg(edges_hbm, out_hbm, dst_v, ones_v, zbuf, acc, ssem):
        cid = lax.axis_index("c")
        sid = lax.axis_index("s")
        wid = cid * NS + sid

        def fill(k, _):
            zbuf[pl.ds(k * 16, 16)] = jnp.zeros((16,), F32)
            return 0

        lax.fori_loop(0, ROWS_PER_TILE // 16, fill, 0)

        def fill1(k, _):
            ones_v[pl.ds(k * 16, 16)] = jnp.ones((16,), F32)
            return 0

        lax.fori_loop(0, CH // 16, fill1, 0)

        row0 = sid * ROWS_PER_TILE
        pltpu.sync_copy(zbuf, acc.at[pl.ds(row0, ROWS_PER_TILE)])
        pltpu.sync_copy(edges_hbm.at[1, wid], dst_v)

        plsc.subcore_barrier()

        def chunk(j, _):
            pltpu.sync_copy(ones_v, acc.at[dst_v.at[j]], add=True)
            return 0

        lax.fori_loop(0, NCHD, chunk, 0)

        plsc.subcore_barrier()
        pltpu.sync_copy(acc.at[pl.ds(row0, ROWS_PER_TILE)],
                        out_hbm.at[cid, pl.ds(row0, ROWS_PER_TILE)])

    return deg


def _matmul_plain(x, w):
    din = x.shape[1]
    dout = w.shape[1]

    def body(h_ref, w_ref, o_ref):
        o_ref[...] = jnp.dot(h_ref[...], w_ref[...],
                             preferred_element_type=F32)

    return pl.pallas_call(
        body,
        grid=(N_PAD // BR,),
        in_specs=[
            pl.BlockSpec((BR, din), lambda i: (i, 0)),
            pl.BlockSpec((din, dout), lambda i: (0, 0)),
        ],
        out_specs=pl.BlockSpec((BR, dout), lambda i: (i, 0)),
        out_shape=jax.ShapeDtypeStruct((N_PAD, dout), F32),
    )(x, w)


def _scale_first(z, degs_t):
    dout = z.shape[1]
    hd = dout // 2

    def body(z_ref, dg_ref, o_ref, dv_ref):
        dinv = lax.rsqrt(dg_ref[:, 0:1] + dg_ref[:, 1:2] + 1.0)
        zs = dinv * z_ref[...]
        o_ref[0] = zs[:, :hd]
        o_ref[1] = zs[:, hd:]
        dv_ref[...] = dinv

    return pl.pallas_call(
        body,
        grid=(N_PAD // BR,),
        in_specs=[
            pl.BlockSpec((BR, dout), lambda i: (i, 0)),
            pl.BlockSpec((BR, NC), lambda i: (i, 0)),
        ],
        out_specs=[
            pl.BlockSpec((NC, BR, hd), lambda i: (0, i, 0)),
            pl.BlockSpec((BR, 1), lambda i: (i, 0)),
        ],
        out_shape=[
            jax.ShapeDtypeStruct((NC, N_PAD, hd), F32),
            jax.ShapeDtypeStruct((N_PAD, 1), F32),
        ],
    )(z, degs_t)


def _matmul_next(acc, zs2, dinv_col, b_row, w):
    hd_in = zs2.shape[2]
    din = 2 * hd_in
    dout = w.shape[1]
    hd = dout // 2

    def body(a_ref, z_ref, dv_ref, b_ref, w_ref, o_ref):
        dv = dv_ref[...]
        h_lo = jnp.maximum(dv * (a_ref[0] + z_ref[0]) + b_ref[:, :hd_in], 0.0)
        h_hi = jnp.maximum(dv * (a_ref[1] + z_ref[1]) + b_ref[:, hd_in:], 0.0)
        h = jnp.concatenate([h_lo, h_hi], axis=1)
        zs = dv * jnp.dot(h, w_ref[...], preferred_element_type=F32)
        o_ref[0] = zs[:, :hd]
        o_ref[1] = zs[:, hd:]

    return pl.pallas_call(
        body,
        grid=(N_PAD // BR,),
        in_specs=[
            pl.BlockSpec((NC, BR, hd_in), lambda i: (0, i, 0)),
            pl.BlockSpec((NC, BR, hd_in), lambda i: (0, i, 0)),
            pl.BlockSpec((BR, 1), lambda i: (i, 0)),
            pl.BlockSpec((1, din), lambda i: (0, 0)),
            pl.BlockSpec((din, dout), lambda i: (0, 0)),
        ],
        out_specs=pl.BlockSpec((NC, BR, hd), lambda i: (0, i, 0)),
        out_shape=jax.ShapeDtypeStruct((NC, N_PAD, hd), F32),
    )(acc, zs2, dinv_col, b_row, w)


def _pool_head(acc, zs2, dinv_col, b_row, batch_col, wl, bl_row):
    hd_in = zs2.shape[2]
    d = 2 * hd_in
    out_d = wl.shape[1]
    nblk = N_PAD // BR

    def body(a_ref, z_ref, dv_ref, b_ref, bt_ref, wl_ref, bl_ref, o_ref,
             sums, counts):
        i = pl.program_id(0)

        @pl.when(i == 0)
        def _():
            sums[...] = jnp.zeros_like(sums)
            counts[...] = jnp.zeros_like(counts)

        dv = dv_ref[...]
        h_lo = jnp.maximum(dv * (a_ref[0] + z_ref[0]) + b_ref[:, :hd_in], 0.0)
        h_hi = jnp.maximum(dv * (a_ref[1] + z_ref[1]) + b_ref[:, hd_in:], 0.0)
        h = jnp.concatenate([h_lo, h_hi], axis=1)

        gids = lax.broadcasted_iota(jnp.int32, (BR, G), 1)
        oh = (bt_ref[...] == gids).astype(F32)
        dn = (((0,), (0,)), ((), ()))
        sums[...] += lax.dot_general(oh, h, dn, preferred_element_type=F32)
        counts[...] += lax.dot_general(oh, jnp.ones((BR, 1), F32), dn,
                                       preferred_element_type=F32)

        @pl.when(i == nblk - 1)
        def _():
            pooled = sums[...] / jnp.maximum(counts[...], 1.0)
            o_ref[...] = (jnp.dot(pooled, wl_ref[...],
                                  preferred_element_type=F32) + bl_ref[...])

    return pl.pallas_call(
        body,
        grid=(nblk,),
        in_specs=[
            pl.BlockSpec((NC, BR, hd_in), lambda i: (0, i, 0)),
            pl.BlockSpec((NC, BR, hd_in), lambda i: (0, i, 0)),
            pl.BlockSpec((BR, 1), lambda i: (i, 0)),
            pl.BlockSpec((1, d), lambda i: (0, 0)),
            pl.BlockSpec((BR, 1), lambda i: (i, 0)),
            pl.BlockSpec(wl.shape, lambda i: (0, 0)),
            pl.BlockSpec((1, out_d), lambda i: (0, 0)),
        ],
        out_specs=pl.BlockSpec((G, out_d), lambda i: (0, 0)),
        out_shape=jax.ShapeDtypeStruct((G, out_d), F32),
        scratch_shapes=[
            pltpu.VMEM((G, d), F32),
            pltpu.VMEM((G, 1), F32),
        ],
    )(acc, zs2, dinv_col, b_row, batch_col, wl, bl_row)


def kernel(x, edge_index, batch, W1, b1, W2, b2, W3, b3, Wl, bl):
    xp = jnp.pad(x, ((0, N_PAD - N), (0, 0)))
    edges_r = edge_index.reshape(2, NS, NCHP, CHP)
    edges_d = edge_index.reshape(2, NC * NS, NCHD, CH)
    batch_col = jnp.pad(batch, (0, N_PAD - N),
                        constant_values=G).reshape(N_PAD, 1)

    degs_t = _make_deg()(edges_d).T
    z1 = _matmul_plain(xp, W1)

    zs2, dinv_col = _scale_first(z1, degs_t)
    acc = _make_seg_sum(W1.shape[1])(zs2, edges_r)

    zs2 = _matmul_next(acc, zs2, dinv_col, b1.reshape(1, -1), W2)
    acc = _make_seg_sum(W2.shape[1])(zs2, edges_r)

    zs2 = _matmul_next(acc, zs2, dinv_col, b2.reshape(1, -1), W3)
    acc = _make_seg_sum(W3.shape[1])(zs2, edges_r)

    out = _pool_head(acc, zs2, dinv_col, b3.reshape(1, -1), batch_col,
                     Wl, bl.reshape(1, -1))
    return out.reshape(G, 192, 16)


# speedup vs baseline: 1.0273x; 1.0273x over previous
import functools

import jax
import jax.numpy as jnp
from jax import lax
from jax.experimental import pallas as pl
from jax.experimental.pallas import tpu as pltpu, tpu_sc as plsc

N = 10000
E = 320000
G = 64
N_PAD = 10240
NC = 2
NS = 16
CH = 80
EPT = E // NS
CHP = 80
NCHP = 250
NCHD = E // (NC * NS) // CH
NBUF = 5
ROWS_PER_TILE = N_PAD // NS
BR = 256
F32 = jnp.float32


def _mesh():
    return plsc.VectorSubcoreMesh(core_axis_name="c", subcore_axis_name="s")


def _zero_fill(ref, rows, d):
    per_row = d // 16

    def body(k, _):
        i = k // per_row
        j = k % per_row
        ref[i, pl.ds(j * 16, 16)] = jnp.zeros((16,), F32)
        return 0

    lax.fori_loop(0, rows * per_row, body, 0)


def _make_seg_sum(d):
    hd = d // 2
    zrows = 64
    nz = ROWS_PER_TILE // zrows
    nbuf = {64: 5, 32: 25, 16: 25}[hd]

    @functools.partial(
        pl.kernel,
        mesh=_mesh(),
        compiler_params=pltpu.CompilerParams(use_tc_tiling_on_sc=False),
        out_type=jax.ShapeDtypeStruct((NC, N_PAD, hd), F32),
        scratch_types=[
            pltpu.VMEM((NCHP, CHP), jnp.int32),
            pltpu.VMEM((NCHP, CHP), jnp.int32),
            pltpu.VMEM((nbuf * CHP, hd), F32),
            pltpu.VMEM((zrows, hd), F32),
            pltpu.VMEM_SHARED((N_PAD, hd), F32),
            pltpu.SemaphoreType.DMA,
            pltpu.SemaphoreType.DMA,
        ],
    )
    def seg(zs2_hbm, edges_hbm, out_hbm, src_v, dst_v, rows_v, zbuf, acc,
            gsem, ssem):
        cid = lax.axis_index("c")
        sid = lax.axis_index("s")
        row0 = sid * ROWS_PER_TILE

        _zero_fill(zbuf, zrows, d // 2)
        for i in range(nz):
            pltpu.sync_copy(zbuf, acc.at[pl.ds(row0 + i * zrows, zrows)])
        pltpu.sync_copy(edges_hbm.at[0, sid], src_v)
        pltpu.sync_copy(edges_hbm.at[1, sid], dst_v)

        plsc.subcore_barrier()

        zsrc = zs2_hbm.at[cid]

        def group(g, _):
            j0 = g * nbuf
            for b in range(nbuf):
                pltpu.async_copy(zsrc.at[src_v.at[j0 + b]],
                                 rows_v.at[pl.ds(b * CHP, CHP)], gsem)
            pltpu.make_async_copy(zsrc.at[pl.ds(0, nbuf * CHP)], rows_v,
                                  gsem).wait()
            for b in range(nbuf):
                pltpu.async_copy(rows_v.at[pl.ds(b * CHP, CHP)],
                                 acc.at[dst_v.at[j0 + b]], ssem, add=True)
            pltpu.make_async_copy(rows_v, acc.at[pl.ds(0, nbuf * CHP)],
                                  ssem).wait()
            return 0

        lax.fori_loop(0, NCHP // nbuf, group, 0)

        plsc.subcore_barrier()
        pltpu.sync_copy(acc.at[pl.ds(row0, ROWS_PER_TILE)],
                        out_hbm.at[cid, pl.ds(row0, ROWS_PER_TILE)])

    return seg


def _make_deg():
    @functools.partial(
        pl.kernel,
        mesh=_mesh(),
        compiler_params=pltpu.CompilerParams(use_tc_tiling_on_sc=False),
        out_type=jax.ShapeDtypeStruct((NC, N_PAD), F32),
        scratch_types=[
            pltpu.VMEM((NCHD, CH), jnp.int32),
            pltpu.VMEM((CH,), F32),
            pltpu.VMEM((ROWS_PER_TILE,), F32),
            pltpu.VMEM_SHARED((N_PAD,), F32),
            pltpu.SemaphoreType.DMA((NBUF,)),
        ],
    )
    def deg(edges_hbm, out_hbm, dst_v, ones_v, zbuf, acc, ssem):
        cid = lax.axis_index("c")
        sid = lax.axis_index("s")
        wid = cid * NS + sid

        def fill(k, _):
            zbuf[pl.ds(k * 16, 16)] = jnp.zeros((16,), F32)
            return 0

        lax.fori_loop(0, ROWS_PER_TILE // 16, fill, 0)

        def fill1(k, _):
            ones_v[pl.ds(k * 16, 16)] = jnp.ones((16,), F32)
            return 0

        lax.fori_loop(0, CH // 16, fill1, 0)

        row0 = sid * ROWS_PER_TILE
        pltpu.sync_copy(zbuf, acc.at[pl.ds(row0, ROWS_PER_TILE)])
        pltpu.sync_copy(edges_hbm.at[1, wid], dst_v)

        plsc.subcore_barrier()

        def chunk(j, _):
            pltpu.sync_copy(ones_v, acc.at[dst_v.at[j]], add=True)
            return 0

        lax.fori_loop(0, NCHD, chunk, 0)

        plsc.subcore_barrier()
        pltpu.sync_copy(acc.at[pl.ds(row0, ROWS_PER_TILE)],
                        out_hbm.at[cid, pl.ds(row0, ROWS_PER_TILE)])

    return deg


def _matmul_first(x, w, degs_t):
    din = x.shape[1]
    dout = w.shape[1]
    hd = dout // 2

    def body(h_ref, w_ref, dg_ref, o_ref, dv_ref):
        dinv = lax.rsqrt(dg_ref[:, 0:1] + dg_ref[:, 1:2] + 1.0)
        z = jnp.dot(h_ref[...], w_ref[...], preferred_element_type=F32)
        zs = dinv * z
        o_ref[0] = zs[:, :hd]
        o_ref[1] = zs[:, hd:]
        dv_ref[...] = dinv

    return pl.pallas_call(
        body,
        grid=(N_PAD // BR,),
        in_specs=[
            pl.BlockSpec((BR, din), lambda i: (i, 0)),
            pl.BlockSpec((din, dout), lambda i: (0, 0)),
            pl.BlockSpec((BR, NC), lambda i: (i, 0)),
        ],
        out_specs=[
            pl.BlockSpec((NC, BR, hd), lambda i: (0, i, 0)),
            pl.BlockSpec((BR, 1), lambda i: (i, 0)),
        ],
        out_shape=[
            jax.ShapeDtypeStruct((NC, N_PAD, hd), F32),
            jax.ShapeDtypeStruct((N_PAD, 1), F32),
        ],
    )(x, w, degs_t)


def _matmul_next(acc, zs2, dinv_col, b_row, w):
    hd_in = zs2.shape[2]
    din = 2 * hd_in
    dout = w.shape[1]
    hd = dout // 2

    def body(a_ref, z_ref, dv_ref, b_ref, w_ref, o_ref):
        dv = dv_ref[...]
        h_lo = jnp.maximum(dv * (a_ref[0] + z_ref[0]) + b_ref[:, :hd_in], 0.0)
        h_hi = jnp.maximum(dv * (a_ref[1] + z_ref[1]) + b_ref[:, hd_in:], 0.0)
        h = jnp.concatenate([h_lo, h_hi], axis=1)
        zs = dv * jnp.dot(h, w_ref[...], preferred_element_type=F32)
        o_ref[0] = zs[:, :hd]
        o_ref[1] = zs[:, hd:]

    return pl.pallas_call(
        body,
        grid=(N_PAD // BR,),
        in_specs=[
            pl.BlockSpec((NC, BR, hd_in), lambda i: (0, i, 0)),
            pl.BlockSpec((NC, BR, hd_in), lambda i: (0, i, 0)),
            pl.BlockSpec((BR, 1), lambda i: (i, 0)),
            pl.BlockSpec((1, din), lambda i: (0, 0)),
            pl.BlockSpec((din, dout), lambda i: (0, 0)),
        ],
        out_specs=pl.BlockSpec((NC, BR, hd), lambda i: (0, i, 0)),
        out_shape=jax.ShapeDtypeStruct((NC, N_PAD, hd), F32),
    )(acc, zs2, dinv_col, b_row, w)


def _pool_head(acc, zs2, dinv_col, b_row, batch_col, wl, bl_row):
    hd_in = zs2.shape[2]
    d = 2 * hd_in
    out_d = wl.shape[1]
    nblk = N_PAD // BR

    def body(a_ref, z_ref, dv_ref, b_ref, bt_ref, wl_ref, bl_ref, o_ref,
             sums, counts):
        i = pl.program_id(0)

        @pl.when(i == 0)
        def _():
            sums[...] = jnp.zeros_like(sums)
            counts[...] = jnp.zeros_like(counts)

        dv = dv_ref[...]
        h_lo = jnp.maximum(dv * (a_ref[0] + z_ref[0]) + b_ref[:, :hd_in], 0.0)
        h_hi = jnp.maximum(dv * (a_ref[1] + z_ref[1]) + b_ref[:, hd_in:], 0.0)
        h = jnp.concatenate([h_lo, h_hi], axis=1)

        gids = lax.broadcasted_iota(jnp.int32, (BR, G), 1)
        oh = (bt_ref[...] == gids).astype(F32)
        dn = (((0,), (0,)), ((), ()))
        sums[...] += lax.dot_general(oh, h, dn, preferred_element_type=F32)
        counts[...] += lax.dot_general(oh, jnp.ones((BR, 1), F32), dn,
                                       preferred_element_type=F32)

        @pl.when(i == nblk - 1)
        def _():
            pooled = sums[...] / jnp.maximum(counts[...], 1.0)
            o_ref[...] = (jnp.dot(pooled, wl_ref[...],
                                  preferred_element_type=F32) + bl_ref[...])

    return pl.pallas_call(
        body,
        grid=(nblk,),
        in_specs=[
            pl.BlockSpec((NC, BR, hd_in), lambda i: (0, i, 0)),
            pl.BlockSpec((NC, BR, hd_in), lambda i: (0, i, 0)),
            pl.BlockSpec((BR, 1), lambda i: (i, 0)),
            pl.BlockSpec((1, d), lambda i: (0, 0)),
            pl.BlockSpec((BR, 1), lambda i: (i, 0)),
            pl.BlockSpec(wl.shape, lambda i: (0, 0)),
            pl.BlockSpec((1, out_d), lambda i: (0, 0)),
        ],
        out_specs=pl.BlockSpec((G, out_d), lambda i: (0, 0)),
        out_shape=jax.ShapeDtypeStruct((G, out_d), F32),
        scratch_shapes=[
            pltpu.VMEM((G, d), F32),
            pltpu.VMEM((G, 1), F32),
        ],
    )(acc, zs2, dinv_col, b_row, batch_col, wl, bl_row)


def kernel(x, edge_index, batch, W1, b1, W2, b2, W3, b3, Wl, bl):
    xp = jnp.pad(x, ((0, N_PAD - N), (0, 0)))
    edges_r = edge_index.reshape(2, NS, NCHP, CHP)
    edges_d = edge_index.reshape(2, NC * NS, NCHD, CH)
    batch_col = jnp.pad(batch, (0, N_PAD - N),
                        constant_values=G).reshape(N_PAD, 1)

    degs_t = _make_deg()(edges_d).T

    zs2, dinv_col = _matmul_first(xp, W1, degs_t)
    acc = _make_seg_sum(W1.shape[1])(zs2, edges_r)

    zs2 = _matmul_next(acc, zs2, dinv_col, b1.reshape(1, -1), W2)
    acc = _make_seg_sum(W2.shape[1])(zs2, edges_r)

    zs2 = _matmul_next(acc, zs2, dinv_col, b2.reshape(1, -1), W3)
    acc = _make_seg_sum(W3.shape[1])(zs2, edges_r)

    out = _pool_head(acc, zs2, dinv_col, b3.reshape(1, -1), batch_col,
                     Wl, bl.reshape(1, -1))
    return out.reshape(G, 192, 16)


# speedup vs baseline: 1.0420x; 1.0143x over previous
import functools

import jax
import jax.numpy as jnp
from jax import lax
from jax.experimental import pallas as pl
from jax.experimental.pallas import tpu as pltpu, tpu_sc as plsc

N = 10000
E = 320000
G = 64
N_PAD = 10240
NC = 2
NS = 16
CH = 80
EPT = E // NS
CHP = 80
NCHP = 250
NCHD = E // (NC * NS) // CH
NBUF = 5
ROWS_PER_TILE = N_PAD // NS
BR = 256
F32 = jnp.float32


def _mesh():
    return plsc.VectorSubcoreMesh(core_axis_name="c", subcore_axis_name="s")


def _zero_fill(ref, rows, d):
    per_row = d // 16

    def body(k, _):
        i = k // per_row
        j = k % per_row
        ref[i, pl.ds(j * 16, 16)] = jnp.zeros((16,), F32)
        return 0

    lax.fori_loop(0, rows * per_row, body, 0)


def _make_seg_sum(d):
    hd = d // 2
    zrows = 64
    nz = ROWS_PER_TILE // zrows
    nbuf = {64: 10, 32: 25, 16: 25}[hd]
    npiece = {64: 5, 32: 1, 16: 1}[hd]
    pc = NCHP // npiece

    @functools.partial(
        pl.kernel,
        mesh=_mesh(),
        compiler_params=pltpu.CompilerParams(use_tc_tiling_on_sc=False),
        out_type=jax.ShapeDtypeStruct((NC, N_PAD, hd), F32),
        scratch_types=[
            pltpu.VMEM((pc, CHP), jnp.int32),
            pltpu.VMEM((pc, CHP), jnp.int32),
            pltpu.VMEM((nbuf * CHP, hd), F32),
            pltpu.VMEM((zrows, hd), F32),
            pltpu.VMEM_SHARED((N_PAD, hd), F32),
            pltpu.SemaphoreType.DMA,
            pltpu.SemaphoreType.DMA,
        ],
    )
    def seg(zs2_hbm, edges_hbm, out_hbm, src_v, dst_v, rows_v, zbuf, acc,
            gsem, ssem):
        cid = lax.axis_index("c")
        sid = lax.axis_index("s")
        row0 = sid * ROWS_PER_TILE

        _zero_fill(zbuf, zrows, d // 2)
        for i in range(nz):
            pltpu.sync_copy(zbuf, acc.at[pl.ds(row0 + i * zrows, zrows)])

        plsc.subcore_barrier()

        zsrc = zs2_hbm.at[cid]

        def group(g, _):
            j0 = g * nbuf
            for b in range(nbuf):
                pltpu.async_copy(zsrc.at[src_v.at[j0 + b]],
                                 rows_v.at[pl.ds(b * CHP, CHP)], gsem)
            pltpu.make_async_copy(zsrc.at[pl.ds(0, nbuf * CHP)], rows_v,
                                  gsem).wait()
            for b in range(nbuf):
                pltpu.async_copy(rows_v.at[pl.ds(b * CHP, CHP)],
                                 acc.at[dst_v.at[j0 + b]], ssem, add=True)
            pltpu.make_async_copy(rows_v, acc.at[pl.ds(0, nbuf * CHP)],
                                  ssem).wait()
            return 0

        for piece in range(npiece):
            if npiece == 1:
                pltpu.sync_copy(edges_hbm.at[0, sid], src_v)
                pltpu.sync_copy(edges_hbm.at[1, sid], dst_v)
            else:
                pltpu.sync_copy(edges_hbm.at[0, sid, piece], src_v)
                pltpu.sync_copy(edges_hbm.at[1, sid, piece], dst_v)
            lax.fori_loop(0, pc // nbuf, group, 0)

        plsc.subcore_barrier()
        pltpu.sync_copy(acc.at[pl.ds(row0, ROWS_PER_TILE)],
                        out_hbm.at[cid, pl.ds(row0, ROWS_PER_TILE)])

    return seg


def _make_deg():
    @functools.partial(
        pl.kernel,
        mesh=_mesh(),
        compiler_params=pltpu.CompilerParams(use_tc_tiling_on_sc=False),
        out_type=jax.ShapeDtypeStruct((NC, N_PAD), F32),
        scratch_types=[
            pltpu.VMEM((NCHD, CH), jnp.int32),
            pltpu.VMEM((CH,), F32),
            pltpu.VMEM((ROWS_PER_TILE,), F32),
            pltpu.VMEM_SHARED((N_PAD,), F32),
            pltpu.SemaphoreType.DMA((NBUF,)),
        ],
    )
    def deg(edges_hbm, out_hbm, dst_v, ones_v, zbuf, acc, ssem):
        cid = lax.axis_index("c")
        sid = lax.axis_index("s")
        wid = cid * NS + sid

        def fill(k, _):
            zbuf[pl.ds(k * 16, 16)] = jnp.zeros((16,), F32)
            return 0

        lax.fori_loop(0, ROWS_PER_TILE // 16, fill, 0)

        def fill1(k, _):
            ones_v[pl.ds(k * 16, 16)] = jnp.ones((16,), F32)
            return 0

        lax.fori_loop(0, CH // 16, fill1, 0)

        row0 = sid * ROWS_PER_TILE
        pltpu.sync_copy(zbuf, acc.at[pl.ds(row0, ROWS_PER_TILE)])
        pltpu.sync_copy(edges_hbm.at[1, wid], dst_v)

        plsc.subcore_barrier()

        def chunk(j, _):
            pltpu.sync_copy(ones_v, acc.at[dst_v.at[j]], add=True)
            return 0

        lax.fori_loop(0, NCHD, chunk, 0)

        plsc.subcore_barrier()
        pltpu.sync_copy(acc.at[pl.ds(row0, ROWS_PER_TILE)],
                        out_hbm.at[cid, pl.ds(row0, ROWS_PER_TILE)])

    return deg


def _matmul_first(x, w, degs_t):
    din = x.shape[1]
    dout = w.shape[1]
    hd = dout // 2

    def body(h_ref, w_ref, dg_ref, o_ref, dv_ref):
        dinv = lax.rsqrt(dg_ref[:, 0:1] + dg_ref[:, 1:2] + 1.0)
        z = jnp.dot(h_ref[...], w_ref[...], preferred_element_type=F32)
        zs = dinv * z
        o_ref[0] = zs[:, :hd]
        o_ref[1] = zs[:, hd:]
        dv_ref[...] = dinv

    return pl.pallas_call(
        body,
        grid=(N_PAD // BR,),
        in_specs=[
            pl.BlockSpec((BR, din), lambda i: (i, 0)),
            pl.BlockSpec((din, dout), lambda i: (0, 0)),
            pl.BlockSpec((BR, NC), lambda i: (i, 0)),
        ],
        out_specs=[
            pl.BlockSpec((NC, BR, hd), lambda i: (0, i, 0)),
            pl.BlockSpec((BR, 1), lambda i: (i, 0)),
        ],
        out_shape=[
            jax.ShapeDtypeStruct((NC, N_PAD, hd), F32),
            jax.ShapeDtypeStruct((N_PAD, 1), F32),
        ],
    )(x, w, degs_t)


def _matmul_next(acc, zs2, dinv_col, b_row, w):
    hd_in = zs2.shape[2]
    din = 2 * hd_in
    dout = w.shape[1]
    hd = dout // 2

    def body(a_ref, z_ref, dv_ref, b_ref, w_ref, o_ref):
        dv = dv_ref[...]
        h_lo = jnp.maximum(dv * (a_ref[0] + z_ref[0]) + b_ref[:, :hd_in], 0.0)
        h_hi = jnp.maximum(dv * (a_ref[1] + z_ref[1]) + b_ref[:, hd_in:], 0.0)
        h = jnp.concatenate([h_lo, h_hi], axis=1)
        zs = dv * jnp.dot(h, w_ref[...], preferred_element_type=F32)
        o_ref[0] = zs[:, :hd]
        o_ref[1] = zs[:, hd:]

    return pl.pallas_call(
        body,
        grid=(N_PAD // BR,),
        in_specs=[
            pl.BlockSpec((NC, BR, hd_in), lambda i: (0, i, 0)),
            pl.BlockSpec((NC, BR, hd_in), lambda i: (0, i, 0)),
            pl.BlockSpec((BR, 1), lambda i: (i, 0)),
            pl.BlockSpec((1, din), lambda i: (0, 0)),
            pl.BlockSpec((din, dout), lambda i: (0, 0)),
        ],
        out_specs=pl.BlockSpec((NC, BR, hd), lambda i: (0, i, 0)),
        out_shape=jax.ShapeDtypeStruct((NC, N_PAD, hd), F32),
    )(acc, zs2, dinv_col, b_row, w)


def _pool_head(acc, zs2, dinv_col, b_row, batch_col, wl, bl_row):
    hd_in = zs2.shape[2]
    d = 2 * hd_in
    out_d = wl.shape[1]
    nblk = N_PAD // BR

    def body(a_ref, z_ref, dv_ref, b_ref, bt_ref, wl_ref, bl_ref, o_ref,
             sums, counts):
        i = pl.program_id(0)

        @pl.when(i == 0)
        def _():
            sums[...] = jnp.zeros_like(sums)
            counts[...] = jnp.zeros_like(counts)

        dv = dv_ref[...]
        h_lo = jnp.maximum(dv * (a_ref[0] + z_ref[0]) + b_ref[:, :hd_in], 0.0)
        h_hi = jnp.maximum(dv * (a_ref[1] + z_ref[1]) + b_ref[:, hd_in:], 0.0)
        h = jnp.concatenate([h_lo, h_hi], axis=1)

        gids = lax.broadcasted_iota(jnp.int32, (BR, G), 1)
        oh = (bt_ref[...] == gids).astype(F32)
        dn = (((0,), (0,)), ((), ()))
        sums[...] += lax.dot_general(oh, h, dn, preferred_element_type=F32)
        counts[...] += lax.dot_general(oh, jnp.ones((BR, 1), F32), dn,
                                       preferred_element_type=F32)

        @pl.when(i == nblk - 1)
        def _():
            pooled = sums[...] / jnp.maximum(counts[...], 1.0)
            o_ref[...] = (jnp.dot(pooled, wl_ref[...],
                                  preferred_element_type=F32) + bl_ref[...])

    return pl.pallas_call(
        body,
        grid=(nblk,),
        in_specs=[
            pl.BlockSpec((NC, BR, hd_in), lambda i: (0, i, 0)),
            pl.BlockSpec((NC, BR, hd_in), lambda i: (0, i, 0)),
            pl.BlockSpec((BR, 1), lambda i: (i, 0)),
            pl.BlockSpec((1, d), lambda i: (0, 0)),
            pl.BlockSpec((BR, 1), lambda i: (i, 0)),
            pl.BlockSpec(wl.shape, lambda i: (0, 0)),
            pl.BlockSpec((1, out_d), lambda i: (0, 0)),
        ],
        out_specs=pl.BlockSpec((G, out_d), lambda i: (0, 0)),
        out_shape=jax.ShapeDtypeStruct((G, out_d), F32),
        scratch_shapes=[
            pltpu.VMEM((G, d), F32),
            pltpu.VMEM((G, 1), F32),
        ],
    )(acc, zs2, dinv_col, b_row, batch_col, wl, bl_row)


def kernel(x, edge_index, batch, W1, b1, W2, b2, W3, b3, Wl, bl):
    xp = jnp.pad(x, ((0, N_PAD - N), (0, 0)))
    edges_r = edge_index.reshape(2, NS, NCHP, CHP)
    edges_r5 = edge_index.reshape(2, NS, 5, NCHP // 5, CHP)
    edges_d = edge_index.reshape(2, NC * NS, NCHD, CH)
    batch_col = jnp.pad(batch, (0, N_PAD - N),
                        constant_values=G).reshape(N_PAD, 1)

    degs_t = _make_deg()(edges_d).T

    zs2, dinv_col = _matmul_first(xp, W1, degs_t)
    acc = _make_seg_sum(W1.shape[1])(zs2, edges_r5)

    zs2 = _matmul_next(acc, zs2, dinv_col, b1.reshape(1, -1), W2)
    acc = _make_seg_sum(W2.shape[1])(zs2, edges_r)

    zs2 = _matmul_next(acc, zs2, dinv_col, b2.reshape(1, -1), W3)
    acc = _make_seg_sum(W3.shape[1])(zs2, edges_r)

    out = _pool_head(acc, zs2, dinv_col, b3.reshape(1, -1), batch_col,
                     Wl, bl.reshape(1, -1))
    return out.reshape(G, 192, 16)
